# baseline (speedup 1.0000x reference)
import functools

import numpy as np

import jax
import jax.numpy as jnp
from jax import lax
from jax.experimental import pallas as pl
from jax.experimental.pallas import tpu as pltpu
from jax.experimental.pallas import tpu_sc as plsc

N = 10000
E = 320000
D_IN = 128
HID = 64
NUM_CLASSES = 4
NUM_GRAPHS = 64

NC = 2
NS = 16
NW = NC * NS

BLK = 128
NB = 80
EP = NW * NB * BLK
NP = 10112
TS = NP // NS

_HIGH = jax.lax.Precision.DEFAULT
_mesh = plsc.VectorSubcoreMesh(core_axis_name="c", subcore_axis_name="s")


def _wid():
  return lax.axis_index("s") * NC + lax.axis_index("c")


def _deg_kernel_body(dst_hbm, out_hbm, dstv, ones_v, zero_v, deg_sh, sem):
  cid = lax.axis_index("c")
  sid = lax.axis_index("s")
  w = _wid()

  @pl.loop(0, 8)
  def _(i):
    ones_v[pl.ds(i * 16, 16), :] = jnp.ones((16, 16), jnp.float32)
    zero_v[pl.ds(i * 16, 16), :] = jnp.zeros((16, 16), jnp.float32)

  @pl.loop(0, 4)
  def _(i):
    pltpu.sync_copy(zero_v, deg_sh.at[pl.ds(sid * TS + i * 128, 128)])
  pltpu.sync_copy(zero_v.at[pl.ds(0, TS - 512)],
                  deg_sh.at[pl.ds(sid * TS + 512, TS - 512)])

  pltpu.sync_copy(dst_hbm.at[pl.ds(w * NB, NB)], dstv)
  plsc.subcore_barrier()

  @pl.loop(0, NB)
  def _(j):
    pltpu.async_copy(ones_v, deg_sh.at[dstv.at[j]], sem, add=True)

    @pl.when(j >= 8)
    def _():
      pltpu.make_async_copy(ones_v, deg_sh.at[dstv.at[0]], sem).wait()

  for _i in range(8):
    pltpu.make_async_copy(ones_v, deg_sh.at[dstv.at[0]], sem).wait()
  plsc.subcore_barrier()
  pltpu.sync_copy(deg_sh.at[pl.ds(sid * TS, TS)],
                  out_hbm.at[cid, pl.ds(sid * TS, TS)])


@jax.jit
def _deg_call(dst2):
  f = pl.kernel(
      _deg_kernel_body,
      out_type=jax.ShapeDtypeStruct((NC, NP, 16), jnp.float32),
      mesh=_mesh,
      compiler_params=pltpu.CompilerParams(use_tc_tiling_on_sc=False),
      scratch_types=[
          pltpu.VMEM((NB, BLK), jnp.int32),
          pltpu.VMEM((BLK, 16), jnp.float32),
          pltpu.VMEM((BLK, 16), jnp.float32),
          pltpu.VMEM_SHARED((NP, 16), jnp.float32),
          pltpu.SemaphoreType.DMA,
      ],
  )
  return f(dst2)


NBUF = 8
LEAD = 4


def _agg_kernel_body(y_hbm, src_hbm, dst_hbm, out_hbm, srcv, dstv,
                     r0, r1, r2, r3, r4, r5, r6, r7,
                     z_sh,
                     g0, g1, g2, g3, g4, g5, g6, g7,
                     s0, s1, s2, s3, s4, s5, s6, s7):
  rows = (r0, r1, r2, r3, r4, r5, r6, r7)
  sg = (g0, g1, g2, g3, g4, g5, g6, g7)
  ss = (s0, s1, s2, s3, s4, s5, s6, s7)
  cid = lax.axis_index("c")
  sid = lax.axis_index("s")
  w = _wid()

  @pl.loop(0, 32)
  def _(i):
    r0[pl.ds(i * 4, 4), :] = jnp.zeros((4, HID), jnp.float32)

  @pl.loop(0, 4)
  def _(i):
    pltpu.sync_copy(r0, z_sh.at[pl.ds(sid * TS + i * BLK, BLK)])
  pltpu.sync_copy(r0.at[pl.ds(0, TS - 4 * BLK)],
                  z_sh.at[pl.ds(sid * TS + 4 * BLK, TS - 4 * BLK)])

  pltpu.sync_copy(src_hbm.at[pl.ds(w * NB, NB)], srcv)
  pltpu.sync_copy(dst_hbm.at[pl.ds(w * NB, NB)], dstv)
  plsc.subcore_barrier()

  for b in range(LEAD):
    pltpu.async_copy(y_hbm.at[srcv.at[b]], rows[b], sg[b])

  @pl.loop(0, NB, step=NBUF)
  def _(z):
    for bb in range(NBUF):
      x = z + bb
      pltpu.make_async_copy(y_hbm.at[srcv.at[x]], rows[bb], sg[bb]).wait()
      pltpu.async_copy(rows[bb], z_sh.at[dstv.at[x]], ss[bb], add=True)
      rb = (bb + LEAD) % NBUF
      r = x + LEAD

      @pl.when((x >= LEAD) & (r < NB))
      def _():
        pltpu.make_async_copy(rows[rb], z_sh.at[dstv.at[0]], ss[rb]).wait()
        pltpu.async_copy(y_hbm.at[srcv.at[r]], rows[rb], sg[rb])

      @pl.when((x < LEAD) & (r < NB))
      def _():
        pltpu.async_copy(y_hbm.at[srcv.at[r]], rows[rb], sg[rb])

  for bb in range(NBUF):
    pltpu.make_async_copy(rows[bb], z_sh.at[dstv.at[0]], ss[bb]).wait()
  plsc.subcore_barrier()
  pltpu.sync_copy(z_sh.at[pl.ds(sid * TS, TS)],
                  out_hbm.at[cid, pl.ds(sid * TS, TS)])


@jax.jit
def _agg_call(y, src2, dst2):
  f = pl.kernel(
      _agg_kernel_body,
      out_type=jax.ShapeDtypeStruct((NC, NP, HID), jnp.float32),
      mesh=_mesh,
      compiler_params=pltpu.CompilerParams(use_tc_tiling_on_sc=False),
      scratch_types=(
          [pltpu.VMEM((NB, BLK), jnp.int32)] * 2
          + [pltpu.VMEM((BLK, HID), jnp.float32)] * 8
          + [pltpu.VMEM_SHARED((NP, HID), jnp.float32)]
          + [pltpu.SemaphoreType.DMA] * 16
      ),
  )
  return f(y, src2, dst2)


_RB = 1000
_NG = N // _RB


def _combine1_body(degp_ref, x_ref, w1_ref, xw_ref, y_ref, dinv_ref):
  deg = degp_ref[0, :, 0:1] + degp_ref[1, :, 0:1] + 1.0
  dinv = lax.rsqrt(deg)
  xw = jnp.dot(x_ref[...], w1_ref[...], preferred_element_type=jnp.float32,
               precision=_HIGH)
  xw_ref[...] = xw
  y_ref[...] = xw * dinv
  dinv_ref[...] = dinv


@jax.jit
def _combine1(degp, x, W1):
  return pl.pallas_call(
      _combine1_body,
      grid=(_NG,),
      in_specs=[
          pl.BlockSpec((2, _RB, 16), lambda i: (0, i, 0)),
          pl.BlockSpec((_RB, D_IN), lambda i: (i, 0)),
          pl.BlockSpec((D_IN, HID), lambda i: (0, 0)),
      ],
      out_specs=[
          pl.BlockSpec((_RB, HID), lambda i: (i, 0)),
          pl.BlockSpec((_RB, HID), lambda i: (i, 0)),
          pl.BlockSpec((_RB, 1), lambda i: (i, 0)),
      ],
      out_shape=[
          jax.ShapeDtypeStruct((N, HID), jnp.float32),
          jax.ShapeDtypeStruct((N, HID), jnp.float32),
          jax.ShapeDtypeStruct((N, 1), jnp.float32),
      ],
  )(degp, x, W1)


def _combine2_body(zp_ref, xw1_ref, dinv_ref, b1_ref, w2_ref,
                   xw2_ref, y2_ref):
  dinv = dinv_ref[...]
  z = zp_ref[0] + zp_ref[1]
  h1 = jnp.maximum(z * dinv + xw1_ref[...] * (dinv * dinv) + b1_ref[...], 0.0)
  xw2 = jnp.dot(h1, w2_ref[...], preferred_element_type=jnp.float32,
                precision=_HIGH)
  xw2_ref[...] = xw2
  y2_ref[...] = xw2 * dinv


@jax.jit
def _combine2(z1p, xw1, dinv, b1r, W2):
  return pl.pallas_call(
      _combine2_body,
      grid=(_NG,),
      in_specs=[
          pl.BlockSpec((2, _RB, HID), lambda i: (0, i, 0)),
          pl.BlockSpec((_RB, HID), lambda i: (i, 0)),
          pl.BlockSpec((_RB, 1), lambda i: (i, 0)),
          pl.BlockSpec((1, HID), lambda i: (0, 0)),
          pl.BlockSpec((HID, HID), lambda i: (0, 0)),
      ],
      out_specs=[
          pl.BlockSpec((_RB, HID), lambda i: (i, 0)),
          pl.BlockSpec((_RB, HID), lambda i: (i, 0)),
      ],
      out_shape=[
          jax.ShapeDtypeStruct((N, HID), jnp.float32),
          jax.ShapeDtypeStruct((N, HID), jnp.float32),
      ],
  )(z1p, xw1, dinv, b1r, W2)


def _head_body(zp_ref, xw2_ref, dinv_ref, b2_ref, batch_ref,
               wfc1_ref, bfc1_ref, wfc2_ref, bfc2_ref, out_ref):
  dinv = dinv_ref[...]
  z = zp_ref[0] + zp_ref[1]
  h2 = jnp.maximum(z * dinv + xw2_ref[...] * (dinv * dinv) + b2_ref[...], 0.0)
  segt = (batch_ref[...] ==
          lax.broadcasted_iota(jnp.int32, (NUM_GRAPHS, N), 0)
          ).astype(jnp.float32)
  sums = jnp.dot(segt, h2, preferred_element_type=jnp.float32,
                 precision=_HIGH)
  counts = jnp.sum(segt, axis=1)[:, None]
  pooled = sums / jnp.maximum(counts, 1.0)
  hf = jnp.maximum(
      jnp.dot(pooled, wfc1_ref[...], preferred_element_type=jnp.float32,
              precision=_HIGH) + bfc1_ref[...], 0.0)
  out_ref[...] = jnp.dot(hf, wfc2_ref[...],
                         preferred_element_type=jnp.float32,
                         precision=_HIGH) + bfc2_ref[...]


@jax.jit
def _head(z2p, xw2, dinv, b2r, batch_r, Wfc1, bfc1r, Wfc2, bfc2r):
  return pl.pallas_call(
      _head_body,
      grid=(1,),
      in_specs=[
          pl.BlockSpec((2, N, HID), lambda i: (0, 0, 0)),
          pl.no_block_spec,
          pl.no_block_spec,
          pl.no_block_spec,
          pl.no_block_spec,
          pl.no_block_spec,
          pl.no_block_spec,
          pl.no_block_spec,
          pl.no_block_spec,
      ],
      out_specs=pl.BlockSpec((NUM_GRAPHS, NUM_CLASSES), lambda i: (0, 0)),
      out_shape=jax.ShapeDtypeStruct((NUM_GRAPHS, NUM_CLASSES), jnp.float32),
  )(z2p, xw2, dinv, b2r, batch_r, Wfc1, bfc1r, Wfc2, bfc2r)


_AR = np.arange(EP - E, dtype=np.int32)
_PAD_SRC = _AR % BLK
_PAD_DST = (N + (_AR % (NP - N))).astype(np.int32)


def kernel(x, edge_index, batch, W1, b1, W2, b2, Wfc1, bfc1, Wfc2, bfc2):
  src = edge_index[0]
  dst = edge_index[1]
  src2 = jnp.concatenate([src, _PAD_SRC]).reshape(EP // BLK, BLK)
  dst2 = jnp.concatenate([dst, _PAD_DST]).reshape(EP // BLK, BLK)

  degp = _deg_call(dst2)
  xw1, y1, dinv = _combine1(degp, x, W1)
  z1p = _agg_call(y1, src2, dst2)
  xw2, y2 = _combine2(z1p, xw1, dinv, b1.reshape(1, HID), W2)
  z2p = _agg_call(y2, src2, dst2)
  out = _head(z2p, xw2, dinv, b2.reshape(1, HID), batch.reshape(1, N),
              Wfc1, bfc1.reshape(1, 128), Wfc2,
              bfc2.reshape(1, NUM_CLASSES))
  return out


# speedup vs baseline: 1.2282x; 1.2282x over previous
import functools

import numpy as np

import jax
import jax.numpy as jnp
from jax import lax
from jax.experimental import pallas as pl
from jax.experimental.pallas import tpu as pltpu
from jax.experimental.pallas import tpu_sc as plsc

N = 10000
E = 320000
D_IN = 128
HID = 64
NUM_CLASSES = 4
NUM_GRAPHS = 64

NC = 2
NS = 16
NW = NC * NS

BLK = 128
NB = 80
EP = NW * NB * BLK
NP = 10112
TS = NP // NS

_HIGH = jax.lax.Precision.DEFAULT
_mesh = plsc.VectorSubcoreMesh(core_axis_name="c", subcore_axis_name="s")


def _wid():
  return lax.axis_index("s") * NC + lax.axis_index("c")


def _deg_kernel_body(dst_hbm, out_hbm, dstv, ones_v, zero_v, deg_sh, sem):
  cid = lax.axis_index("c")
  sid = lax.axis_index("s")
  w = _wid()

  @pl.loop(0, 8)
  def _(i):
    ones_v[pl.ds(i * 16, 16), :] = jnp.ones((16, 16), jnp.float32)
    zero_v[pl.ds(i * 16, 16), :] = jnp.zeros((16, 16), jnp.float32)

  @pl.loop(0, 4)
  def _(i):
    pltpu.sync_copy(zero_v, deg_sh.at[pl.ds(sid * TS + i * 128, 128)])
  pltpu.sync_copy(zero_v.at[pl.ds(0, TS - 512)],
                  deg_sh.at[pl.ds(sid * TS + 512, TS - 512)])

  pltpu.sync_copy(dst_hbm.at[pl.ds(w * NB, NB)], dstv)
  plsc.subcore_barrier()

  @pl.loop(0, NB)
  def _(j):
    pltpu.async_copy(ones_v, deg_sh.at[dstv.at[j]], sem, add=True)

    @pl.when(j >= 8)
    def _():
      pltpu.make_async_copy(ones_v, deg_sh.at[dstv.at[0]], sem).wait()

  for _i in range(8):
    pltpu.make_async_copy(ones_v, deg_sh.at[dstv.at[0]], sem).wait()
  plsc.subcore_barrier()
  pltpu.sync_copy(deg_sh.at[pl.ds(sid * TS, TS)],
                  out_hbm.at[cid, pl.ds(sid * TS, TS)])


@jax.jit
def _deg_call(dst2):
  f = pl.kernel(
      _deg_kernel_body,
      out_type=jax.ShapeDtypeStruct((NC, NP, 16), jnp.float32),
      mesh=_mesh,
      compiler_params=pltpu.CompilerParams(use_tc_tiling_on_sc=False),
      scratch_types=[
          pltpu.VMEM((NB, BLK), jnp.int32),
          pltpu.VMEM((BLK, 16), jnp.float32),
          pltpu.VMEM((BLK, 16), jnp.float32),
          pltpu.VMEM_SHARED((NP, 16), jnp.float32),
          pltpu.SemaphoreType.DMA,
      ],
  )
  return f(dst2)


NBUF = 8
LEAD = 4


def _agg_kernel_body(y_hbm, src_hbm, dst_hbm, out_hbm, srcv, dstv,
                     r0, r1, r2, r3, r4, r5, r6, r7,
                     z_sh,
                     g0, g1, g2, g3, g4, g5, g6, g7,
                     s0, s1, s2, s3, s4, s5, s6, s7):
  rows = (r0, r1, r2, r3, r4, r5, r6, r7)
  sg = (g0, g1, g2, g3, g4, g5, g6, g7)
  ss = (s0, s1, s2, s3, s4, s5, s6, s7)
  cid = lax.axis_index("c")
  sid = lax.axis_index("s")
  w = _wid()

  @pl.loop(0, 32)
  def _(i):
    r0[pl.ds(i * 4, 4), :] = jnp.zeros((4, HID), jnp.bfloat16)

  @pl.loop(0, 4)
  def _(i):
    pltpu.sync_copy(r0, z_sh.at[pl.ds(sid * TS + i * BLK, BLK)])
  pltpu.sync_copy(r0.at[pl.ds(0, TS - 4 * BLK)],
                  z_sh.at[pl.ds(sid * TS + 4 * BLK, TS - 4 * BLK)])

  pltpu.sync_copy(src_hbm.at[pl.ds(w * NB, NB)], srcv)
  pltpu.sync_copy(dst_hbm.at[pl.ds(w * NB, NB)], dstv)
  plsc.subcore_barrier()

  for b in range(LEAD):
    pltpu.async_copy(y_hbm.at[srcv.at[b]], rows[b], sg[b])

  @pl.loop(0, NB, step=NBUF)
  def _(z):
    for bb in range(NBUF):
      x = z + bb
      pltpu.make_async_copy(y_hbm.at[srcv.at[x]], rows[bb], sg[bb]).wait()
      pltpu.async_copy(rows[bb], z_sh.at[dstv.at[x]], ss[bb], add=True)
      rb = (bb + LEAD) % NBUF
      r = x + LEAD

      @pl.when((x >= LEAD) & (r < NB))
      def _():
        pltpu.make_async_copy(rows[rb], z_sh.at[dstv.at[0]], ss[rb]).wait()
        pltpu.async_copy(y_hbm.at[srcv.at[r]], rows[rb], sg[rb])

      @pl.when((x < LEAD) & (r < NB))
      def _():
        pltpu.async_copy(y_hbm.at[srcv.at[r]], rows[rb], sg[rb])

  for bb in range(NBUF):
    pltpu.make_async_copy(rows[bb], z_sh.at[dstv.at[0]], ss[bb]).wait()
  plsc.subcore_barrier()
  pltpu.sync_copy(z_sh.at[pl.ds(sid * TS, TS)],
                  out_hbm.at[cid, pl.ds(sid * TS, TS)])


@jax.jit
def _agg_call(y, src2, dst2):
  f = pl.kernel(
      _agg_kernel_body,
      out_type=jax.ShapeDtypeStruct((NC, NP, HID), jnp.bfloat16),
      mesh=_mesh,
      compiler_params=pltpu.CompilerParams(use_tc_tiling_on_sc=False),
      scratch_types=(
          [pltpu.VMEM((NB, BLK), jnp.int32)] * 2
          + [pltpu.VMEM((BLK, HID), jnp.bfloat16)] * 8
          + [pltpu.VMEM_SHARED((NP, HID), jnp.bfloat16)]
          + [pltpu.SemaphoreType.DMA] * 16
      ),
  )
  return f(y, src2, dst2)


_RB = 1000
_NG = N // _RB


def _combine1_body(degp_ref, x_ref, w1_ref, xw_ref, y_ref, dinv_ref):
  deg = degp_ref[0, :, 0:1] + degp_ref[1, :, 0:1] + 1.0
  dinv = lax.rsqrt(deg)
  xw = jnp.dot(x_ref[...], w1_ref[...], preferred_element_type=jnp.float32,
               precision=_HIGH)
  xw_ref[...] = xw
  y_ref[...] = (xw * dinv).astype(jnp.bfloat16)
  dinv_ref[...] = dinv


@jax.jit
def _combine1(degp, x, W1):
  return pl.pallas_call(
      _combine1_body,
      grid=(_NG,),
      in_specs=[
          pl.BlockSpec((2, _RB, 16), lambda i: (0, i, 0)),
          pl.BlockSpec((_RB, D_IN), lambda i: (i, 0)),
          pl.BlockSpec((D_IN, HID), lambda i: (0, 0)),
      ],
      out_specs=[
          pl.BlockSpec((_RB, HID), lambda i: (i, 0)),
          pl.BlockSpec((_RB, HID), lambda i: (i, 0)),
          pl.BlockSpec((_RB, 1), lambda i: (i, 0)),
      ],
      out_shape=[
          jax.ShapeDtypeStruct((N, HID), jnp.float32),
          jax.ShapeDtypeStruct((N, HID), jnp.bfloat16),
          jax.ShapeDtypeStruct((N, 1), jnp.float32),
      ],
  )(degp, x, W1)


def _combine2_body(zp_ref, xw1_ref, dinv_ref, b1_ref, w2_ref,
                   xw2_ref, y2_ref):
  dinv = dinv_ref[...]
  z = zp_ref[0].astype(jnp.float32) + zp_ref[1].astype(jnp.float32)
  h1 = jnp.maximum(z * dinv + xw1_ref[...] * (dinv * dinv) + b1_ref[...], 0.0)
  xw2 = jnp.dot(h1, w2_ref[...], preferred_element_type=jnp.float32,
                precision=_HIGH)
  xw2_ref[...] = xw2
  y2_ref[...] = (xw2 * dinv).astype(jnp.bfloat16)


@jax.jit
def _combine2(z1p, xw1, dinv, b1r, W2):
  return pl.pallas_call(
      _combine2_body,
      grid=(_NG,),
      in_specs=[
          pl.BlockSpec((2, _RB, HID), lambda i: (0, i, 0)),
          pl.BlockSpec((_RB, HID), lambda i: (i, 0)),
          pl.BlockSpec((_RB, 1), lambda i: (i, 0)),
          pl.BlockSpec((1, HID), lambda i: (0, 0)),
          pl.BlockSpec((HID, HID), lambda i: (0, 0)),
      ],
      out_specs=[
          pl.BlockSpec((_RB, HID), lambda i: (i, 0)),
          pl.BlockSpec((_RB, HID), lambda i: (i, 0)),
      ],
      out_shape=[
          jax.ShapeDtypeStruct((N, HID), jnp.float32),
          jax.ShapeDtypeStruct((N, HID), jnp.bfloat16),
      ],
  )(z1p, xw1, dinv, b1r, W2)


def _head_body(zp_ref, xw2_ref, dinv_ref, b2_ref, batch_ref,
               wfc1_ref, bfc1_ref, wfc2_ref, bfc2_ref, out_ref):
  dinv = dinv_ref[...]
  z = zp_ref[0].astype(jnp.float32) + zp_ref[1].astype(jnp.float32)
  h2 = jnp.maximum(z * dinv + xw2_ref[...] * (dinv * dinv) + b2_ref[...], 0.0)
  segt = (batch_ref[...] ==
          lax.broadcasted_iota(jnp.int32, (NUM_GRAPHS, N), 0)
          ).astype(jnp.float32)
  sums = jnp.dot(segt, h2, preferred_element_type=jnp.float32,
                 precision=_HIGH)
  counts = jnp.sum(segt, axis=1)[:, None]
  pooled = sums / jnp.maximum(counts, 1.0)
  hf = jnp.maximum(
      jnp.dot(pooled, wfc1_ref[...], preferred_element_type=jnp.float32,
              precision=_HIGH) + bfc1_ref[...], 0.0)
  out_ref[...] = jnp.dot(hf, wfc2_ref[...],
                         preferred_element_type=jnp.float32,
                         precision=_HIGH) + bfc2_ref[...]


@jax.jit
def _head(z2p, xw2, dinv, b2r, batch_r, Wfc1, bfc1r, Wfc2, bfc2r):
  return pl.pallas_call(
      _head_body,
      grid=(1,),
      in_specs=[
          pl.BlockSpec((2, N, HID), lambda i: (0, 0, 0)),
          pl.no_block_spec,
          pl.no_block_spec,
          pl.no_block_spec,
          pl.no_block_spec,
          pl.no_block_spec,
          pl.no_block_spec,
          pl.no_block_spec,
          pl.no_block_spec,
      ],
      out_specs=pl.BlockSpec((NUM_GRAPHS, NUM_CLASSES), lambda i: (0, 0)),
      out_shape=jax.ShapeDtypeStruct((NUM_GRAPHS, NUM_CLASSES), jnp.float32),
  )(z2p, xw2, dinv, b2r, batch_r, Wfc1, bfc1r, Wfc2, bfc2r)


_AR = np.arange(EP - E, dtype=np.int32)
_PAD_SRC = _AR % BLK
_PAD_DST = (N + (_AR % (NP - N))).astype(np.int32)


def kernel(x, edge_index, batch, W1, b1, W2, b2, Wfc1, bfc1, Wfc2, bfc2):
  src = edge_index[0]
  dst = edge_index[1]
  src2 = jnp.concatenate([src, _PAD_SRC]).reshape(EP // BLK, BLK)
  dst2 = jnp.concatenate([dst, _PAD_DST]).reshape(EP // BLK, BLK)

  degp = _deg_call(dst2)
  xw1, y1, dinv = _combine1(degp, x, W1)
  z1p = _agg_call(y1, src2, dst2)
  xw2, y2 = _combine2(z1p, xw1, dinv, b1.reshape(1, HID), W2)
  z2p = _agg_call(y2, src2, dst2)
  out = _head(z2p, xw2, dinv, b2.reshape(1, HID), batch.reshape(1, N),
              Wfc1, bfc1.reshape(1, 128), Wfc2,
              bfc2.reshape(1, NUM_CLASSES))
  return out


# speedup vs baseline: 1.2532x; 1.0203x over previous
import functools

import numpy as np

import jax
import jax.numpy as jnp
from jax import lax
from jax.experimental import pallas as pl
from jax.experimental.pallas import tpu as pltpu
from jax.experimental.pallas import tpu_sc as plsc

N = 10000
E = 320000
D_IN = 128
HID = 64
NUM_CLASSES = 4
NUM_GRAPHS = 64

NC = 2
NS = 16
NW = NC * NS

BLK = 128
NB = 80
EP = NW * NB * BLK
NP = 10112
TS = NP // NS

_HIGH = jax.lax.Precision.DEFAULT
_mesh = plsc.VectorSubcoreMesh(core_axis_name="c", subcore_axis_name="s")


def _wid():
  return lax.axis_index("s") * NC + lax.axis_index("c")


def _deg_kernel_body(dst_hbm, out_hbm, dstv, ones_v, zero_v, deg_sh, sem):
  cid = lax.axis_index("c")
  sid = lax.axis_index("s")
  w = _wid()

  @pl.loop(0, 8)
  def _(i):
    ones_v[pl.ds(i * 16, 16), :] = jnp.ones((16, 16), jnp.float32)
    zero_v[pl.ds(i * 16, 16), :] = jnp.zeros((16, 16), jnp.float32)

  @pl.loop(0, 4)
  def _(i):
    pltpu.sync_copy(zero_v, deg_sh.at[pl.ds(sid * TS + i * 128, 128)])
  pltpu.sync_copy(zero_v.at[pl.ds(0, TS - 512)],
                  deg_sh.at[pl.ds(sid * TS + 512, TS - 512)])

  pltpu.sync_copy(dst_hbm.at[pl.ds(w * NB, NB)], dstv)
  plsc.subcore_barrier()

  @pl.loop(0, NB)
  def _(j):
    pltpu.async_copy(ones_v, deg_sh.at[dstv.at[j]], sem, add=True)

    @pl.when(j >= 8)
    def _():
      pltpu.make_async_copy(ones_v, deg_sh.at[dstv.at[0]], sem).wait()

  for _i in range(8):
    pltpu.make_async_copy(ones_v, deg_sh.at[dstv.at[0]], sem).wait()
  plsc.subcore_barrier()
  pltpu.sync_copy(deg_sh.at[pl.ds(sid * TS, TS)],
                  out_hbm.at[cid, pl.ds(sid * TS, TS)])


@jax.jit
def _deg_call(dst2):
  f = pl.kernel(
      _deg_kernel_body,
      out_type=jax.ShapeDtypeStruct((NC, NP, 16), jnp.float32),
      mesh=_mesh,
      compiler_params=pltpu.CompilerParams(use_tc_tiling_on_sc=False),
      scratch_types=[
          pltpu.VMEM((NB, BLK), jnp.int32),
          pltpu.VMEM((BLK, 16), jnp.float32),
          pltpu.VMEM((BLK, 16), jnp.float32),
          pltpu.VMEM_SHARED((NP, 16), jnp.float32),
          pltpu.SemaphoreType.DMA,
      ],
  )
  return f(dst2)


NBUF = 8
LEAD = 4


def _agg_kernel_body(y_hbm, src_hbm, dst_hbm, out_hbm, srcv, dstv,
                     r0, r1, r2, r3, r4, r5, r6, r7,
                     z_sh,
                     g0, g1, g2, g3, g4, g5, g6, g7,
                     s0, s1, s2, s3, s4, s5, s6, s7):
  rows = (r0, r1, r2, r3, r4, r5, r6, r7)
  sg = (g0, g1, g2, g3, g4, g5, g6, g7)
  ss = (s0, s1, s2, s3, s4, s5, s6, s7)
  cid = lax.axis_index("c")
  sid = lax.axis_index("s")
  w = _wid()

  @pl.loop(0, 32)
  def _(i):
    r0[pl.ds(i * 4, 4), :] = jnp.zeros((4, HID), jnp.bfloat16)

  @pl.loop(0, 4)
  def _(i):
    pltpu.sync_copy(r0, z_sh.at[pl.ds(sid * TS + i * BLK, BLK)])
  pltpu.sync_copy(r0.at[pl.ds(0, TS - 4 * BLK)],
                  z_sh.at[pl.ds(sid * TS + 4 * BLK, TS - 4 * BLK)])

  pltpu.sync_copy(src_hbm.at[pl.ds(w * NB, NB)], srcv)
  pltpu.sync_copy(dst_hbm.at[pl.ds(w * NB, NB)], dstv)
  plsc.subcore_barrier()

  for b in range(LEAD):
    pltpu.async_copy(y_hbm.at[srcv.at[b]], rows[b], sg[b])

  @pl.loop(0, NB, step=NBUF)
  def _(z):
    for bb in range(NBUF):
      x = z + bb
      pltpu.make_async_copy(y_hbm.at[srcv.at[x]], rows[bb], sg[bb]).wait()
      pltpu.async_copy(rows[bb], z_sh.at[dstv.at[x]], ss[bb], add=True)
      rb = (bb + LEAD) % NBUF
      r = x + LEAD

      @pl.when((x >= LEAD) & (r < NB))
      def _():
        pltpu.make_async_copy(rows[rb], z_sh.at[dstv.at[0]], ss[rb]).wait()
        pltpu.async_copy(y_hbm.at[srcv.at[r]], rows[rb], sg[rb])

      @pl.when((x < LEAD) & (r < NB))
      def _():
        pltpu.async_copy(y_hbm.at[srcv.at[r]], rows[rb], sg[rb])

  for bb in range(NBUF):
    pltpu.make_async_copy(rows[bb], z_sh.at[dstv.at[0]], ss[bb]).wait()
  plsc.subcore_barrier()
  pltpu.sync_copy(z_sh.at[pl.ds(sid * TS, TS)],
                  out_hbm.at[cid, pl.ds(sid * TS, TS)])


@jax.jit
def _agg_call(y, src2, dst2):
  f = pl.kernel(
      _agg_kernel_body,
      out_type=jax.ShapeDtypeStruct((NC, NP, HID), jnp.bfloat16),
      mesh=_mesh,
      compiler_params=pltpu.CompilerParams(use_tc_tiling_on_sc=False),
      scratch_types=(
          [pltpu.VMEM((NB, BLK), jnp.int32)] * 2
          + [pltpu.VMEM((BLK, HID), jnp.bfloat16)] * 8
          + [pltpu.VMEM_SHARED((NP, HID), jnp.bfloat16)]
          + [pltpu.SemaphoreType.DMA] * 16
      ),
  )
  return f(y, src2, dst2)


_RB = 2000
_NG = N // _RB


def _combine1_body(degp_ref, x_ref, w1_ref, xw_ref, y_ref, dinv_ref):
  deg = degp_ref[0, :, 0:1] + degp_ref[1, :, 0:1] + 1.0
  dinv = lax.rsqrt(deg)
  xw = jnp.dot(x_ref[...], w1_ref[...], preferred_element_type=jnp.float32,
               precision=_HIGH)
  xw_ref[...] = xw
  y_ref[...] = (xw * dinv).astype(jnp.bfloat16)
  dinv_ref[...] = dinv


@jax.jit
def _combine1(degp, x, W1):
  return pl.pallas_call(
      _combine1_body,
      grid=(_NG,),
      in_specs=[
          pl.BlockSpec((2, _RB, 16), lambda i: (0, i, 0)),
          pl.BlockSpec((_RB, D_IN), lambda i: (i, 0)),
          pl.BlockSpec((D_IN, HID), lambda i: (0, 0)),
      ],
      out_specs=[
          pl.BlockSpec((_RB, HID), lambda i: (i, 0)),
          pl.BlockSpec((_RB, HID), lambda i: (i, 0)),
          pl.BlockSpec((_RB, 1), lambda i: (i, 0)),
      ],
      out_shape=[
          jax.ShapeDtypeStruct((N, HID), jnp.float32),
          jax.ShapeDtypeStruct((N, HID), jnp.bfloat16),
          jax.ShapeDtypeStruct((N, 1), jnp.float32),
      ],
  )(degp, x, W1)


def _combine2_body(zp_ref, xw1_ref, dinv_ref, b1_ref, w2_ref,
                   xw2_ref, y2_ref):
  dinv = dinv_ref[...]
  z = zp_ref[0].astype(jnp.float32) + zp_ref[1].astype(jnp.float32)
  h1 = jnp.maximum(z * dinv + xw1_ref[...] * (dinv * dinv) + b1_ref[...], 0.0)
  xw2 = jnp.dot(h1, w2_ref[...], preferred_element_type=jnp.float32,
                precision=_HIGH)
  xw2_ref[...] = xw2
  y2_ref[...] = (xw2 * dinv).astype(jnp.bfloat16)


@jax.jit
def _combine2(z1p, xw1, dinv, b1r, W2):
  return pl.pallas_call(
      _combine2_body,
      grid=(_NG,),
      in_specs=[
          pl.BlockSpec((2, _RB, HID), lambda i: (0, i, 0)),
          pl.BlockSpec((_RB, HID), lambda i: (i, 0)),
          pl.BlockSpec((_RB, 1), lambda i: (i, 0)),
          pl.BlockSpec((1, HID), lambda i: (0, 0)),
          pl.BlockSpec((HID, HID), lambda i: (0, 0)),
      ],
      out_specs=[
          pl.BlockSpec((_RB, HID), lambda i: (i, 0)),
          pl.BlockSpec((_RB, HID), lambda i: (i, 0)),
      ],
      out_shape=[
          jax.ShapeDtypeStruct((N, HID), jnp.float32),
          jax.ShapeDtypeStruct((N, HID), jnp.bfloat16),
      ],
  )(z1p, xw1, dinv, b1r, W2)


def _head_body(zp_ref, xw2_ref, dinv_ref, b2_ref, batch_ref,
               wfc1_ref, bfc1_ref, wfc2_ref, bfc2_ref, out_ref):
  dinv = dinv_ref[...]
  z = zp_ref[0].astype(jnp.float32) + zp_ref[1].astype(jnp.float32)
  h2 = jnp.maximum(z * dinv + xw2_ref[...] * (dinv * dinv) + b2_ref[...], 0.0)
  segt = (batch_ref[...] ==
          lax.broadcasted_iota(jnp.int32, (NUM_GRAPHS, N), 0)
          ).astype(jnp.float32)
  sums = jnp.dot(segt, h2, preferred_element_type=jnp.float32,
                 precision=_HIGH)
  counts = jnp.sum(segt, axis=1)[:, None]
  pooled = sums / jnp.maximum(counts, 1.0)
  hf = jnp.maximum(
      jnp.dot(pooled, wfc1_ref[...], preferred_element_type=jnp.float32,
              precision=_HIGH) + bfc1_ref[...], 0.0)
  out_ref[...] = jnp.dot(hf, wfc2_ref[...],
                         preferred_element_type=jnp.float32,
                         precision=_HIGH) + bfc2_ref[...]


@jax.jit
def _head(z2p, xw2, dinv, b2r, batch_r, Wfc1, bfc1r, Wfc2, bfc2r):
  return pl.pallas_call(
      _head_body,
      grid=(1,),
      in_specs=[
          pl.BlockSpec((2, N, HID), lambda i: (0, 0, 0)),
          pl.no_block_spec,
          pl.no_block_spec,
          pl.no_block_spec,
          pl.no_block_spec,
          pl.no_block_spec,
          pl.no_block_spec,
          pl.no_block_spec,
          pl.no_block_spec,
      ],
      out_specs=pl.BlockSpec((NUM_GRAPHS, NUM_CLASSES), lambda i: (0, 0)),
      out_shape=jax.ShapeDtypeStruct((NUM_GRAPHS, NUM_CLASSES), jnp.float32),
  )(z2p, xw2, dinv, b2r, batch_r, Wfc1, bfc1r, Wfc2, bfc2r)


_AR = np.arange(EP - E, dtype=np.int32)
_PAD_SRC = _AR % BLK
_PAD_DST = (N + (_AR % (NP - N))).astype(np.int32)


def kernel(x, edge_index, batch, W1, b1, W2, b2, Wfc1, bfc1, Wfc2, bfc2):
  src = edge_index[0]
  dst = edge_index[1]
  src2 = jnp.concatenate([src, _PAD_SRC]).reshape(EP // BLK, BLK)
  dst2 = jnp.concatenate([dst, _PAD_DST]).reshape(EP // BLK, BLK)

  degp = _deg_call(dst2)
  xw1, y1, dinv = _combine1(degp, x, W1)
  z1p = _agg_call(y1, src2, dst2)
  xw2, y2 = _combine2(z1p, xw1, dinv, b1.reshape(1, HID), W2)
  z2p = _agg_call(y2, src2, dst2)
  out = _head(z2p, xw2, dinv, b2.reshape(1, HID), batch.reshape(1, N),
              Wfc1, bfc1.reshape(1, 128), Wfc2,
              bfc2.reshape(1, NUM_CLASSES))
  return out


# speedup vs baseline: 1.2558x; 1.0021x over previous
import numpy as np

import jax
import jax.numpy as jnp
from jax import lax
from jax.experimental import pallas as pl
from jax.experimental.pallas import tpu as pltpu
from jax.experimental.pallas import tpu_sc as plsc

N = 10000
E = 320000
D_IN = 128
HID = 64
NUM_CLASSES = 4
NUM_GRAPHS = 64

NC = 2
NS = 16
NW = NC * NS

BLK = 128
NB = 80
EP = NW * NB * BLK
NP = 10112
TS = NP // NS

_HIGH = jax.lax.Precision.DEFAULT
_mesh = plsc.VectorSubcoreMesh(core_axis_name="c", subcore_axis_name="s")


def _wid():
  return lax.axis_index("s") * NC + lax.axis_index("c")


def _deg_kernel_body(dst_hbm, out_hbm, dstv, ones_v, zero_v, deg_sh, sem):
  cid = lax.axis_index("c")
  sid = lax.axis_index("s")
  w = _wid()

  @pl.loop(0, 8)
  def _(i):
    ones_v[pl.ds(i * 16, 16), :] = jnp.ones((16, 16), jnp.float32)
    zero_v[pl.ds(i * 16, 16), :] = jnp.zeros((16, 16), jnp.float32)

  @pl.loop(0, 4)
  def _(i):
    pltpu.sync_copy(zero_v, deg_sh.at[pl.ds(sid * TS + i * 128, 128)])
  pltpu.sync_copy(zero_v.at[pl.ds(0, TS - 512)],
                  deg_sh.at[pl.ds(sid * TS + 512, TS - 512)])

  pltpu.sync_copy(dst_hbm.at[pl.ds(w * NB, NB)], dstv)
  plsc.subcore_barrier()

  @pl.loop(0, NB)
  def _(j):
    pltpu.async_copy(ones_v, deg_sh.at[dstv.at[j]], sem, add=True)

    @pl.when(j >= 8)
    def _():
      pltpu.make_async_copy(ones_v, deg_sh.at[dstv.at[0]], sem).wait()

  for _i in range(8):
    pltpu.make_async_copy(ones_v, deg_sh.at[dstv.at[0]], sem).wait()
  plsc.subcore_barrier()
  pltpu.sync_copy(deg_sh.at[pl.ds(sid * TS, TS)],
                  out_hbm.at[cid, pl.ds(sid * TS, TS)])


@jax.jit
def _deg_call(dst2):
  f = pl.kernel(
      _deg_kernel_body,
      out_type=jax.ShapeDtypeStruct((NC, NP, 16), jnp.float32),
      mesh=_mesh,
      compiler_params=pltpu.CompilerParams(use_tc_tiling_on_sc=False),
      scratch_types=[
          pltpu.VMEM((NB, BLK), jnp.int32),
          pltpu.VMEM((BLK, 16), jnp.float32),
          pltpu.VMEM((BLK, 16), jnp.float32),
          pltpu.VMEM_SHARED((NP, 16), jnp.float32),
          pltpu.SemaphoreType.DMA,
      ],
  )
  return f(dst2)


NBUF = 8
LEAD = 4


def _agg_kernel_body(y_hbm, src_hbm, dst_hbm, out_hbm, srcv, dstv,
                     r0, r1, r2, r3, r4, r5, r6, r7,
                     z_sh,
                     g0, g1, g2, g3, g4, g5, g6, g7,
                     s0, s1, s2, s3, s4, s5, s6, s7):
  rows = (r0, r1, r2, r3, r4, r5, r6, r7)
  sg = (g0, g1, g2, g3, g4, g5, g6, g7)
  ss = (s0, s1, s2, s3, s4, s5, s6, s7)
  cid = lax.axis_index("c")
  sid = lax.axis_index("s")
  w = _wid()

  @pl.loop(0, 32)
  def _(i):
    r0[pl.ds(i * 4, 4), :] = jnp.zeros((4, HID), jnp.bfloat16)

  @pl.loop(0, 4)
  def _(i):
    pltpu.sync_copy(r0, z_sh.at[pl.ds(sid * TS + i * BLK, BLK)])
  pltpu.sync_copy(r0.at[pl.ds(0, TS - 4 * BLK)],
                  z_sh.at[pl.ds(sid * TS + 4 * BLK, TS - 4 * BLK)])

  pltpu.sync_copy(src_hbm.at[pl.ds(w * NB, NB)], srcv)
  pltpu.sync_copy(dst_hbm.at[pl.ds(w * NB, NB)], dstv)
  plsc.subcore_barrier()

  for b in range(LEAD):
    pltpu.async_copy(y_hbm.at[srcv.at[b]], rows[b], sg[b])

  @pl.loop(0, NB, step=NBUF)
  def _(z):
    for bb in range(NBUF):
      x = z + bb
      pltpu.make_async_copy(y_hbm.at[srcv.at[x]], rows[bb], sg[bb]).wait()
      pltpu.async_copy(rows[bb], z_sh.at[dstv.at[x]], ss[bb], add=True)
      rb = (bb + LEAD) % NBUF
      r = x + LEAD

      @pl.when((x >= LEAD) & (r < NB))
      def _():
        pltpu.make_async_copy(rows[rb], z_sh.at[dstv.at[0]], ss[rb]).wait()
        pltpu.async_copy(y_hbm.at[srcv.at[r]], rows[rb], sg[rb])

      @pl.when((x < LEAD) & (r < NB))
      def _():
        pltpu.async_copy(y_hbm.at[srcv.at[r]], rows[rb], sg[rb])

  for bb in range(NBUF):
    pltpu.make_async_copy(rows[bb], z_sh.at[dstv.at[0]], ss[bb]).wait()
  plsc.subcore_barrier()
  pltpu.sync_copy(z_sh.at[pl.ds(sid * TS, TS)],
                  out_hbm.at[cid, pl.ds(sid * TS, TS)])


@jax.jit
def _agg_call(y, src2, dst2):
  f = pl.kernel(
      _agg_kernel_body,
      out_type=jax.ShapeDtypeStruct((NC, NP, HID), jnp.bfloat16),
      mesh=_mesh,
      compiler_params=pltpu.CompilerParams(use_tc_tiling_on_sc=False),
      scratch_types=(
          [pltpu.VMEM((NB, BLK), jnp.int32)] * 2
          + [pltpu.VMEM((BLK, HID), jnp.bfloat16)] * 8
          + [pltpu.VMEM_SHARED((NP, HID), jnp.bfloat16)]
          + [pltpu.SemaphoreType.DMA] * 16
      ),
  )
  return f(y, src2, dst2)


_RB = 2000
_NG = N // _RB


def _combine1_body(degp_ref, x_ref, w1_ref, xw_ref, y_ref, dinv_ref):
  deg = degp_ref[0, :, 0:1] + degp_ref[1, :, 0:1] + 1.0
  dinv = lax.rsqrt(deg)
  xw = jnp.dot(x_ref[...], w1_ref[...], preferred_element_type=jnp.float32,
               precision=_HIGH)
  xw_ref[...] = xw
  y_ref[...] = (xw * dinv).astype(jnp.bfloat16)
  dinv_ref[...] = dinv


@jax.jit
def _combine1(degp, x, W1):
  return pl.pallas_call(
      _combine1_body,
      grid=(_NG,),
      in_specs=[
          pl.BlockSpec((2, _RB, 16), lambda i: (0, i, 0)),
          pl.BlockSpec((_RB, D_IN), lambda i: (i, 0)),
          pl.BlockSpec((D_IN, HID), lambda i: (0, 0)),
      ],
      out_specs=[
          pl.BlockSpec((_RB, HID), lambda i: (i, 0)),
          pl.BlockSpec((_RB, HID), lambda i: (i, 0)),
          pl.BlockSpec((_RB, 1), lambda i: (i, 0)),
      ],
      out_shape=[
          jax.ShapeDtypeStruct((N, HID), jnp.float32),
          jax.ShapeDtypeStruct((N, HID), jnp.bfloat16),
          jax.ShapeDtypeStruct((N, 1), jnp.float32),
      ],
  )(degp, x, W1)


def _combine2_body(zp_ref, xw1_ref, dinv_ref, b1_ref, w2_ref,
                   xw2_ref, y2_ref):
  dinv = dinv_ref[...]
  z = zp_ref[0].astype(jnp.float32) + zp_ref[1].astype(jnp.float32)
  h1 = jnp.maximum(z * dinv + xw1_ref[...] * (dinv * dinv) + b1_ref[...], 0.0)
  xw2 = jnp.dot(h1, w2_ref[...], preferred_element_type=jnp.float32,
                precision=_HIGH)
  xw2_ref[...] = xw2
  y2_ref[...] = (xw2 * dinv).astype(jnp.bfloat16)


@jax.jit
def _combine2(z1p, xw1, dinv, b1r, W2):
  return pl.pallas_call(
      _combine2_body,
      grid=(_NG,),
      in_specs=[
          pl.BlockSpec((2, _RB, HID), lambda i: (0, i, 0)),
          pl.BlockSpec((_RB, HID), lambda i: (i, 0)),
          pl.BlockSpec((_RB, 1), lambda i: (i, 0)),
          pl.BlockSpec((1, HID), lambda i: (0, 0)),
          pl.BlockSpec((HID, HID), lambda i: (0, 0)),
      ],
      out_specs=[
          pl.BlockSpec((_RB, HID), lambda i: (i, 0)),
          pl.BlockSpec((_RB, HID), lambda i: (i, 0)),
      ],
      out_shape=[
          jax.ShapeDtypeStruct((N, HID), jnp.float32),
          jax.ShapeDtypeStruct((N, HID), jnp.bfloat16),
      ],
  )(z1p, xw1, dinv, b1r, W2)


def _head_body(zp_ref, xw2_ref, dinv_ref, b2_ref, batch_ref,
               wfc1_ref, bfc1_ref, wfc2_ref, bfc2_ref, out_ref):
  dinv = dinv_ref[...]
  z = zp_ref[0].astype(jnp.float32) + zp_ref[1].astype(jnp.float32)
  h2 = jnp.maximum(z * dinv + xw2_ref[...] * (dinv * dinv) + b2_ref[...], 0.0)
  segt = (batch_ref[...] ==
          lax.broadcasted_iota(jnp.int32, (NUM_GRAPHS, N), 0)
          ).astype(jnp.float32)
  sums = jnp.dot(segt, h2, preferred_element_type=jnp.float32,
                 precision=_HIGH)
  counts = jnp.sum(segt, axis=1)[:, None]
  pooled = sums / jnp.maximum(counts, 1.0)
  hf = jnp.maximum(
      jnp.dot(pooled, wfc1_ref[...], preferred_element_type=jnp.float32,
              precision=_HIGH) + bfc1_ref[...], 0.0)
  out_ref[...] = jnp.dot(hf, wfc2_ref[...],
                         preferred_element_type=jnp.float32,
                         precision=_HIGH) + bfc2_ref[...]


@jax.jit
def _head(z2p, xw2, dinv, b2r, batch_r, Wfc1, bfc1r, Wfc2, bfc2r):
  return pl.pallas_call(
      _head_body,
      grid=(1,),
      in_specs=[
          pl.BlockSpec((2, N, HID), lambda i: (0, 0, 0)),
          pl.no_block_spec,
          pl.no_block_spec,
          pl.no_block_spec,
          pl.no_block_spec,
          pl.no_block_spec,
          pl.no_block_spec,
          pl.no_block_spec,
          pl.no_block_spec,
      ],
      out_specs=pl.BlockSpec((NUM_GRAPHS, NUM_CLASSES), lambda i: (0, 0)),
      out_shape=jax.ShapeDtypeStruct((NUM_GRAPHS, NUM_CLASSES), jnp.float32),
  )(z2p, xw2, dinv, b2r, batch_r, Wfc1, bfc1r, Wfc2, bfc2r)


_AR = np.arange(EP - E, dtype=np.int32)
_PAD_SRC = _AR % BLK
_PAD_DST = (N + (_AR % (NP - N))).astype(np.int32)


def kernel(x, edge_index, batch, W1, b1, W2, b2, Wfc1, bfc1, Wfc2, bfc2):
  src = edge_index[0]
  dst = edge_index[1]
  src2 = jnp.concatenate([src, _PAD_SRC]).reshape(EP // BLK, BLK)
  dst2 = jnp.concatenate([dst, _PAD_DST]).reshape(EP // BLK, BLK)

  degp = _deg_call(dst2)
  xw1, y1, dinv = _combine1(degp, x, W1)
  z1p = _agg_call(y1, src2, dst2)
  xw2, y2 = _combine2(z1p, xw1, dinv, b1.reshape(1, HID), W2)
  z2p = _agg_call(y2, src2, dst2)
  out = _head(z2p, xw2, dinv, b2.reshape(1, HID), batch.reshape(1, N),
              Wfc1, bfc1.reshape(1, 128), Wfc2,
              bfc2.reshape(1, NUM_CLASSES))
  return out
